# Initial kernel scaffold; baseline (speedup 1.0000x reference)
import jax
import jax.numpy as jnp
from jax.experimental import pallas as pl


def kernel(x, embed_weight):
    raise NotImplementedError("write your pallas kernel here")


# speedup vs baseline: 4.1023x; 4.1023x over previous
import functools

import jax
import jax.numpy as jnp
from jax import lax
from jax.experimental import pallas as pl
from jax.experimental.pallas import tpu as pltpu
from jax.experimental.pallas import tpu_sc as plsc

VOCAB = 100000
EMBED_DIM = 64
BATCH = 4096
HIST = 50

NC = 2
NS = 16
NW = NC * NS

TOTAL = BATCH * HIST
CHUNK = 128
PER_W = TOTAL // NW
N_CHUNKS = PER_W // CHUNK


def _make_kernel():
    mesh = plsc.VectorSubcoreMesh(core_axis_name="c", subcore_axis_name="s")

    @functools.partial(
        pl.kernel,
        out_type=jax.ShapeDtypeStruct((TOTAL, EMBED_DIM), jnp.float32),
        mesh=mesh,
        compiler_params=pltpu.CompilerParams(use_tc_tiling_on_sc=False),
        scratch_types=[
            pltpu.VMEM((PER_W,), jnp.int32),
            pltpu.VMEM((CHUNK, EMBED_DIM), jnp.float32),
            pltpu.SemaphoreType.DMA,
        ],
    )
    def emb_kernel(x_hbm, table_hbm, out_hbm, idx_v, rows_v, sem):
        wid = lax.axis_index("c") * NS + lax.axis_index("s")
        pltpu.sync_copy(x_hbm.at[pl.ds(wid * PER_W, PER_W)], idx_v)

        def body(j, _):
            pltpu.async_copy(
                table_hbm.at[idx_v.at[pl.ds(j * CHUNK, CHUNK)]], rows_v, sem
            ).wait()
            row0 = wid * PER_W + j * CHUNK
            pltpu.sync_copy(rows_v, out_hbm.at[pl.ds(row0, CHUNK)])
            return 0

        lax.fori_loop(0, N_CHUNKS, body, 0)

    return emb_kernel


_emb_kernel = _make_kernel()


@jax.jit
def kernel(x, embed_weight):
    x2 = x.reshape(TOTAL).astype(jnp.int32)
    out = _emb_kernel(x2, embed_weight)
    return out.reshape(BATCH, HIST, EMBED_DIM)


# speedup vs baseline: 4.6204x; 1.1263x over previous
import functools

import jax
import jax.numpy as jnp
from jax import lax
from jax.experimental import pallas as pl
from jax.experimental.pallas import tpu as pltpu
from jax.experimental.pallas import tpu_sc as plsc

VOCAB = 100000
EMBED_DIM = 64
BATCH = 4096
HIST = 50

NC = 2
NS = 16
NW = NC * NS

TOTAL = BATCH * HIST
CHUNK = 128
PER_W = TOTAL // NW
N_CHUNKS = PER_W // CHUNK
NB = 5
OUTER = N_CHUNKS // NB


def _make_kernel():
    mesh = plsc.VectorSubcoreMesh(core_axis_name="c", subcore_axis_name="s")

    @functools.partial(
        pl.kernel,
        out_type=jax.ShapeDtypeStruct((TOTAL, EMBED_DIM), jnp.float32),
        mesh=mesh,
        compiler_params=pltpu.CompilerParams(use_tc_tiling_on_sc=False),
        scratch_types=[
            pltpu.VMEM((PER_W,), jnp.int32),
            pltpu.VMEM((NB, CHUNK, EMBED_DIM), jnp.float32),
            [pltpu.SemaphoreType.DMA] * NB,
            [pltpu.SemaphoreType.DMA] * NB,
        ],
    )
    def emb_kernel(x_hbm, table_hbm, out_hbm, idx_v, rows_v, gsem, wsem):
        wid = lax.axis_index("c") * NS + lax.axis_index("s")
        row_base = wid * PER_W
        pltpu.sync_copy(x_hbm.at[pl.ds(row_base, PER_W)], idx_v)

        def gather_args(j, b):
            return (
                table_hbm.at[idx_v.at[pl.ds(j * CHUNK, CHUNK)]],
                rows_v.at[b],
                gsem[b],
            )

        def write_args(j, b):
            return (
                rows_v.at[b],
                out_hbm.at[pl.ds(row_base + j * CHUNK, CHUNK)],
                wsem[b],
            )

        for b in range(NB):
            pltpu.async_copy(*gather_args(b, b))

        def outer(t, _):
            j0 = t * NB
            for b in range(NB):
                pltpu.make_async_copy(*gather_args(j0 + b, b)).wait()
                pltpu.async_copy(*write_args(j0 + b, b))
            for b in range(NB):
                pltpu.make_async_copy(*write_args(j0 + b, b)).wait()

            @pl.when(t + 1 < OUTER)
            def _():
                for b in range(NB):
                    pltpu.async_copy(*gather_args(j0 + NB + b, b))

            return 0

        lax.fori_loop(0, OUTER, outer, 0)

    return emb_kernel


_emb_kernel = _make_kernel()


@jax.jit
def kernel(x, embed_weight):
    x2 = x.reshape(TOTAL).astype(jnp.int32)
    out = _emb_kernel(x2, embed_weight)
    return out.reshape(BATCH, HIST, EMBED_DIM)
